# baseline (speedup 1.0000x reference)
import functools

import jax
import jax.numpy as jnp
from jax import lax
from jax.experimental import pallas as pl
from jax.experimental.pallas import tpu as pltpu
from jax.experimental.pallas import tpu_sc as plsc

B, T, D = 32, 128, 1024
KD, VD = 64, 64
ES, SS = 16384, 4096
K = 32
LANES = 128
NEG = -3.0e38


def _topk_loop(scores_2d, flat_iota):
    lane = lax.broadcasted_iota(jnp.int32, (1, LANES), 1)
    vals_vec = jnp.full((1, LANES), NEG, dtype=jnp.float32)
    idx_vec = jnp.zeros((1, LANES), dtype=jnp.int32)
    s = scores_2d
    big = jnp.int32(0x7FFFFFFF)
    for k in range(K):
        m = jnp.max(s)
        cand = jnp.where(s == m, flat_iota, big)
        am = jnp.min(cand)
        s = jnp.where(cand == am, NEG, s)
        vals_vec = jnp.where(lane == k, m, vals_vec)
        idx_vec = jnp.where(lane == k, am, idx_vec)
    return vals_vec, idx_vec


def _softmax_weights(vals_vec):
    lane = lax.broadcasted_iota(jnp.int32, (1, LANES), 1)
    valid = (lane < K).astype(jnp.float32)
    w = jnp.exp(vals_vec - jnp.max(vals_vec)) * valid
    return w / jnp.sum(w)


def _score_topk_body(ws_ref, eptr_ref, sptr_ref, hid_ref, q_ref, ek_ref,
                     sk_ref, wk_ref, bk_ref, wv_ref, bv_ref,
                     scores_ref, ew_ref, ei_ref, esub_ref,
                     sw_ref, si_ref, ssub_ref, wvals_ref):
    b = pl.program_id(0)
    mask_b = ws_ref[b] > 0.0
    e_slot = lax.rem(eptr_ref[b], ES)
    s_slot = lax.rem(sptr_ref[b], SS)

    dn_t = (((1,), (1,)), ((), ()))
    dn_n = (((1,), (0,)), ((), ()))

    crep = jnp.mean(hid_ref[0], axis=0, keepdims=True)
    wkeys = lax.dot_general(crep, wk_ref[...], dn_t) + bk_ref[...][None, :]
    wvals = lax.dot_general(crep, wv_ref[...], dn_t) + bv_ref[...][None, :]
    q = lax.dot_general(q_ref[0], wk_ref[...], dn_t) + bk_ref[...][None, :]
    q_dot_wk = jnp.sum(q * wkeys)

    e_row = lax.dot_general(q, ek_ref[0], dn_n)
    col = lax.broadcasted_iota(jnp.int32, (1, ES), 1)
    e_row = jnp.where((col == e_slot) & mask_b, q_dot_wk, e_row)
    scores_ref[0, :, :ES] = e_row
    e2d = e_row.reshape(ES // LANES, LANES)
    eflat = lax.broadcasted_iota(jnp.int32, (ES // LANES, LANES), 0) * LANES \
        + lax.broadcasted_iota(jnp.int32, (ES // LANES, LANES), 1)
    ev, eidx = _topk_loop(e2d, eflat)
    ew_ref[0] = _softmax_weights(ev)
    ei_ref[0] = eidx + b * ES
    esub_ref[0] = jnp.where(
        (eidx == e_slot) & mask_b
        & (lax.broadcasted_iota(jnp.int32, (1, LANES), 1) < K), 1.0, 0.0)

    s_row = lax.dot_general(q, sk_ref[0], dn_n)
    scol = lax.broadcasted_iota(jnp.int32, (1, SS), 1)
    s_row = jnp.where((scol == s_slot) & mask_b, q_dot_wk, s_row)
    scores_ref[0, :, ES:] = s_row
    s2d = s_row.reshape(SS // LANES, LANES)
    sflat = lax.broadcasted_iota(jnp.int32, (SS // LANES, LANES), 0) * LANES \
        + lax.broadcasted_iota(jnp.int32, (SS // LANES, LANES), 1)
    sv, sidx = _topk_loop(s2d, sflat)
    sw_ref[0] = _softmax_weights(sv)
    si_ref[0] = sidx + b * SS
    ssub_ref[0] = jnp.where(
        (sidx == s_slot) & mask_b
        & (lax.broadcasted_iota(jnp.int32, (1, LANES), 1) < K), 1.0, 0.0)

    wvals_ref[0] = wvals


def _score_topk(ws, eptr, sptr, hidden, query, ekT, skT, WkT, bk, WvT, bv):
    f32 = jnp.float32
    i32 = jnp.int32
    small_f = jax.ShapeDtypeStruct((B, 1, LANES), f32)
    small_i = jax.ShapeDtypeStruct((B, 1, LANES), i32)
    small_spec = pl.BlockSpec((1, 1, LANES), lambda b: (b, 0, 0))
    return pl.pallas_call(
        _score_topk_body,
        grid=(B,),
        in_specs=[
            pl.BlockSpec(memory_space=pltpu.SMEM),
            pl.BlockSpec(memory_space=pltpu.SMEM),
            pl.BlockSpec(memory_space=pltpu.SMEM),
            pl.BlockSpec((1, T, D), lambda b: (b, 0, 0)),
            pl.BlockSpec((1, 1, D), lambda b: (b, 0, 0)),
            pl.BlockSpec((1, KD, ES), lambda b: (b, 0, 0)),
            pl.BlockSpec((1, KD, SS), lambda b: (b, 0, 0)),
            pl.BlockSpec((KD, D), lambda b: (0, 0)),
            pl.BlockSpec((KD,), lambda b: (0,)),
            pl.BlockSpec((KD, D), lambda b: (0, 0)),
            pl.BlockSpec((KD,), lambda b: (0,)),
        ],
        out_specs=[
            pl.BlockSpec((1, 1, ES + SS), lambda b: (b, 0, 0)),
            small_spec, small_spec, small_spec,
            small_spec, small_spec, small_spec,
            pl.BlockSpec((1, 1, VD), lambda b: (b, 0, 0)),
        ],
        out_shape=[
            jax.ShapeDtypeStruct((B, 1, ES + SS), f32),
            small_f, small_i, small_f,
            small_f, small_i, small_f,
            jax.ShapeDtypeStruct((B, 1, VD), f32),
        ],
        compiler_params=pltpu.CompilerParams(
            dimension_semantics=("arbitrary",)),
    )(ws, eptr, sptr, hidden, query.reshape(B, 1, D), ekT, skT,
      WkT, bk, WvT, bv)


NW = 32
RPW = (B * K) // NW
EPW = RPW * VD

EB = ES.bit_length() - 1
SB = SS.bit_length() - 1


def _build_addrs(idx_v, addr_v, slot_bits):
    for g in range(RPW // 16):
        gv = idx_v[pl.ds(g * 16, 16)]
        base = ((gv >> slot_bits) << (slot_bits + 6)) \
            + (gv & ((1 << slot_bits) - 1))
        for f in range(VD):
            p = f * RPW + g * 16
            addr_v[p // LANES, pl.ds(p % LANES, 16)] = base + (f << slot_bits)


def _gather_rows(tab_ref, addr_v, rows_v, sem):
    copies = []
    for j in range(EPW // LANES):
        copies.append(pltpu.async_copy(
            tab_ref.at[addr_v.at[j]], rows_v.at[pl.ds(j * LANES, LANES)], sem))
    for c in copies:
        c.wait()


def _sc_gather_body(etab_ref, stab_ref, eidx_ref, sidx_ref,
                    erows_ref, srows_ref, idx_v, addr_v, rows_v, sem):
    wid = lax.axis_index("s") * 2 + lax.axis_index("c")
    base = wid * RPW

    pltpu.sync_copy(eidx_ref.at[pl.ds(base, RPW)], idx_v)
    _build_addrs(idx_v, addr_v, EB)
    _gather_rows(etab_ref, addr_v, rows_v, sem)
    pltpu.sync_copy(rows_v, erows_ref.at[wid])

    pltpu.sync_copy(sidx_ref.at[pl.ds(base, RPW)], idx_v)
    _build_addrs(idx_v, addr_v, SB)
    _gather_rows(stab_ref, addr_v, rows_v, sem)
    pltpu.sync_copy(rows_v, srows_ref.at[wid])


def _sc_gather(etab, stab, eidx, sidx):
    mesh = plsc.VectorSubcoreMesh(core_axis_name="c", subcore_axis_name="s")
    fn = functools.partial(
        pl.kernel,
        out_type=[
            jax.ShapeDtypeStruct((NW, EPW), jnp.float32),
            jax.ShapeDtypeStruct((NW, EPW), jnp.float32),
        ],
        mesh=mesh,
        scratch_types=[
            pltpu.VMEM((RPW,), jnp.int32),
            pltpu.VMEM((EPW // LANES, LANES), jnp.int32),
            pltpu.VMEM((EPW,), jnp.float32),
            pltpu.SemaphoreType.DMA,
        ],
    )(_sc_gather_body)
    return fn(etab, stab, eidx, sidx)


def _combine_body(er_ref, sr_ref, ew_ref, sw_ref, esub_ref, ssub_ref,
                  wv_ref, Wr_ref, br_ref, out_ref):
    wv = wv_ref[...]
    acc = jnp.zeros((B, VD), dtype=jnp.float32)
    for k in range(K):
        er = er_ref[k]
        sr = sr_ref[k]
        es = esub_ref[k][:, None]
        ss_ = ssub_ref[k][:, None]
        ewc = ew_ref[k][:, None]
        swc = sw_ref[k][:, None]
        acc = acc + (er * (1.0 - es) + wv * es) * ewc
        acc = acc + (sr * (1.0 - ss_) + wv * ss_) * swc
    out_ref[...] = jnp.dot(acc, Wr_ref[...]) + br_ref[...][None, :]


def _combine(er_t, sr_t, ew_t, sw_t, esub_t, ssub_t, wvals, Wr, br):
    return pl.pallas_call(
        _combine_body,
        out_shape=jax.ShapeDtypeStruct((B, D), jnp.float32),
    )(er_t, sr_t, ew_t, sw_t, esub_t, ssub_t, wvals, Wr, br)


def kernel(hidden, query, write_scores, episodic_keys, episodic_values,
           semantic_keys, semantic_values, episodic_ptr, semantic_ptr,
           Wk, bk, Wv, bv, Wr, br):
    eptr = episodic_ptr.astype(jnp.int32)
    sptr = semantic_ptr.astype(jnp.int32)

    ekT = jnp.swapaxes(episodic_keys, 1, 2)
    skT = jnp.swapaxes(semantic_keys, 1, 2)
    WkT = Wk.T
    WvT = Wv.T

    (all_scores3, ew, ei, esub, sw, si, ssub, wvals3) = _score_topk(
        write_scores, eptr, sptr, hidden, query, ekT, skT, WkT, bk, WvT, bv)
    all_scores = all_scores3.reshape(B, ES + SS)

    eidx = ei[:, 0, :K].reshape(B * K)
    sidx = si[:, 0, :K].reshape(B * K)
    etab = jnp.swapaxes(episodic_values, 1, 2).reshape(B * VD * ES)
    stab = jnp.swapaxes(semantic_values, 1, 2).reshape(B * VD * SS)
    erows_w, srows_w = _sc_gather(etab, stab, eidx, sidx)

    er_t = erows_w.reshape(B, VD, K).transpose(2, 0, 1)
    sr_t = srows_w.reshape(B, VD, K).transpose(2, 0, 1)
    ew_t = ew[:, 0, :K].T
    sw_t = sw[:, 0, :K].T
    esub_t = esub[:, 0, :K].T
    ssub_t = ssub[:, 0, :K].T
    wvals = wvals3[:, 0, :]

    context = _combine(er_t, sr_t, ew_t, sw_t, esub_t, ssub_t,
                       wvals, Wr, br)
    return context, all_scores


# speedup vs baseline: 3.0366x; 3.0366x over previous
import functools

import jax
import jax.numpy as jnp
from jax import lax
from jax.experimental import pallas as pl
from jax.experimental.pallas import tpu as pltpu
from jax.experimental.pallas import tpu_sc as plsc

B, T, D = 32, 128, 1024
KD, VD = 64, 64
ES, SS = 16384, 4096
K = 32
LANES = 128
NEG = -3.0e38


def _score_body(ws_ref, eptr_ref, sptr_ref, hid_ref, q_ref, ek_ref,
                sk_ref, wk_ref, bk_ref, wv_ref, bv_ref,
                scores_ref, wvals_ref):
    b = pl.program_id(0)
    mask_b = ws_ref[b] > 0.0
    e_slot = lax.rem(eptr_ref[b], ES)
    s_slot = lax.rem(sptr_ref[b], SS)

    dn_t = (((1,), (1,)), ((), ()))
    dn_n = (((1,), (0,)), ((), ()))

    crep = jnp.mean(hid_ref[0], axis=0, keepdims=True)
    wkeys = lax.dot_general(crep, wk_ref[...], dn_t) + bk_ref[...][None, :]
    wvals = lax.dot_general(crep, wv_ref[...], dn_t) + bv_ref[...][None, :]
    q = lax.dot_general(q_ref[0], wk_ref[...], dn_t) + bk_ref[...][None, :]
    q_dot_wk = jnp.sum(q * wkeys)

    e_row = lax.dot_general(q, ek_ref[0], dn_n)
    col = lax.broadcasted_iota(jnp.int32, (1, ES), 1)
    e_row = jnp.where((col == e_slot) & mask_b, q_dot_wk, e_row)
    scores_ref[0, :, :ES] = e_row

    s_row = lax.dot_general(q, sk_ref[0], dn_n)
    scol = lax.broadcasted_iota(jnp.int32, (1, SS), 1)
    s_row = jnp.where((scol == s_slot) & mask_b, q_dot_wk, s_row)
    scores_ref[0, :, ES:] = s_row

    wvals_ref[0] = wvals


def _score(ws, eptr, sptr, hidden, query, ekT, skT, WkT, bk, WvT, bv):
    f32 = jnp.float32
    return pl.pallas_call(
        _score_body,
        grid=(B,),
        in_specs=[
            pl.BlockSpec(memory_space=pltpu.SMEM),
            pl.BlockSpec(memory_space=pltpu.SMEM),
            pl.BlockSpec(memory_space=pltpu.SMEM),
            pl.BlockSpec((1, T, D), lambda b: (b, 0, 0)),
            pl.BlockSpec((1, 1, D), lambda b: (b, 0, 0)),
            pl.BlockSpec((1, KD, ES), lambda b: (b, 0, 0)),
            pl.BlockSpec((1, KD, SS), lambda b: (b, 0, 0)),
            pl.BlockSpec((KD, D), lambda b: (0, 0)),
            pl.BlockSpec((KD,), lambda b: (0,)),
            pl.BlockSpec((KD, D), lambda b: (0, 0)),
            pl.BlockSpec((KD,), lambda b: (0,)),
        ],
        out_specs=[
            pl.BlockSpec((1, 1, ES + SS), lambda b: (b, 0, 0)),
            pl.BlockSpec((1, 1, VD), lambda b: (b, 0, 0)),
        ],
        out_shape=[
            jax.ShapeDtypeStruct((B, 1, ES + SS), f32),
            jax.ShapeDtypeStruct((B, 1, VD), f32),
        ],
        compiler_params=pltpu.CompilerParams(
            dimension_semantics=("arbitrary",)),
    )(ws, eptr, sptr, hidden, query.reshape(B, 1, D), ekT, skT,
      WkT, bk, WvT, bv)


def _topk_all_body(sc_ref, eslot_ref, sslot_ref, maskf_ref,
                   ew_ref, ei_ref, esub_ref, sw_ref, si_ref, ssub_ref):
    lane = lax.broadcasted_iota(jnp.int32, (B, LANES), 1)
    row = lax.broadcasted_iota(jnp.int32, (B, LANES), 0)
    maskb = maskf_ref[...] > 0.5
    big = jnp.int32(0x7FFFFFFF)

    def extract(s, n, slot, base_mul, w_ref, i_ref, sub_ref):
        colio = lax.broadcasted_iota(jnp.int32, (B, n), 1)
        vals = jnp.full((B, LANES), NEG, dtype=jnp.float32)
        idxm = jnp.zeros((B, LANES), dtype=jnp.int32)
        for k in range(K):
            m = jnp.max(s, axis=1, keepdims=True)
            cand = jnp.where(s == m, colio, big)
            am = jnp.min(cand, axis=1, keepdims=True)
            s = jnp.where(cand == am, NEG, s)
            vals = jnp.where(lane == k, m, vals)
            idxm = jnp.where(lane == k, am, idxm)
        valid = (lane < K).astype(jnp.float32)
        w = jnp.exp(vals - vals[:, 0:1]) * valid
        w_ref[...] = w / jnp.sum(w, axis=1, keepdims=True)
        i_ref[...] = idxm + row * base_mul
        sub_ref[...] = jnp.where(
            (idxm == slot) & maskb & (lane < K), 1.0, 0.0)

    extract(sc_ref[:, :ES], ES, eslot_ref[...], ES, ew_ref, ei_ref, esub_ref)
    extract(sc_ref[:, ES:], SS, sslot_ref[...], SS, sw_ref, si_ref, ssub_ref)


def _topk_all(all_scores, eslot, sslot, maskf):
    f32 = jnp.float32
    i32 = jnp.int32
    sf = jax.ShapeDtypeStruct((B, LANES), f32)
    si_ = jax.ShapeDtypeStruct((B, LANES), i32)
    return pl.pallas_call(
        _topk_all_body,
        out_shape=[sf, si_, sf, sf, si_, sf],
    )(all_scores, eslot, sslot, maskf)


NW = 32
RPW = (B * K) // NW
EPW = RPW * VD

EB = ES.bit_length() - 1
SB = SS.bit_length() - 1


def _build_addrs(idx_v, addr_v, slot_bits):
    for g in range(RPW // 16):
        gv = idx_v[pl.ds(g * 16, 16)]
        base = ((gv >> slot_bits) << (slot_bits + 6)) \
            + (gv & ((1 << slot_bits) - 1))
        for f in range(VD):
            p = f * RPW + g * 16
            addr_v[p // LANES, pl.ds(p % LANES, 16)] = base + (f << slot_bits)


def _gather_rows(tab_ref, addr_v, rows_v, sem):
    copies = []
    for j in range(EPW // LANES):
        copies.append(pltpu.async_copy(
            tab_ref.at[addr_v.at[j]], rows_v.at[pl.ds(j * LANES, LANES)], sem))
    for c in copies:
        c.wait()


def _sc_gather_body(etab_ref, stab_ref, eidx_ref, sidx_ref,
                    erows_ref, srows_ref, idx_v, addr_v, rows_v, sem):
    wid = lax.axis_index("s") * 2 + lax.axis_index("c")
    base = wid * RPW

    pltpu.sync_copy(eidx_ref.at[pl.ds(base, RPW)], idx_v)
    _build_addrs(idx_v, addr_v, EB)
    _gather_rows(etab_ref, addr_v, rows_v, sem)
    pltpu.sync_copy(rows_v, erows_ref.at[wid])

    pltpu.sync_copy(sidx_ref.at[pl.ds(base, RPW)], idx_v)
    _build_addrs(idx_v, addr_v, SB)
    _gather_rows(stab_ref, addr_v, rows_v, sem)
    pltpu.sync_copy(rows_v, srows_ref.at[wid])


def _sc_gather(etab, stab, eidx, sidx):
    mesh = plsc.VectorSubcoreMesh(core_axis_name="c", subcore_axis_name="s")
    fn = functools.partial(
        pl.kernel,
        out_type=[
            jax.ShapeDtypeStruct((NW, EPW), jnp.float32),
            jax.ShapeDtypeStruct((NW, EPW), jnp.float32),
        ],
        mesh=mesh,
        scratch_types=[
            pltpu.VMEM((RPW,), jnp.int32),
            pltpu.VMEM((EPW // LANES, LANES), jnp.int32),
            pltpu.VMEM((EPW,), jnp.float32),
            pltpu.SemaphoreType.DMA,
        ],
    )(_sc_gather_body)
    return fn(etab, stab, eidx, sidx)


def _combine_body(er_ref, sr_ref, ew_ref, sw_ref, esub_ref, ssub_ref,
                  wv_ref, Wr_ref, br_ref, out_ref):
    wv = wv_ref[...]
    acc = jnp.zeros((B, VD), dtype=jnp.float32)
    for k in range(K):
        er = er_ref[k]
        sr = sr_ref[k]
        es = esub_ref[k][:, None]
        ss_ = ssub_ref[k][:, None]
        ewc = ew_ref[k][:, None]
        swc = sw_ref[k][:, None]
        acc = acc + (er * (1.0 - es) + wv * es) * ewc
        acc = acc + (sr * (1.0 - ss_) + wv * ss_) * swc
    out_ref[...] = jnp.dot(acc, Wr_ref[...]) + br_ref[...][None, :]


def _combine(er_t, sr_t, ew_t, sw_t, esub_t, ssub_t, wvals, Wr, br):
    return pl.pallas_call(
        _combine_body,
        out_shape=jax.ShapeDtypeStruct((B, D), jnp.float32),
    )(er_t, sr_t, ew_t, sw_t, esub_t, ssub_t, wvals, Wr, br)


def kernel(hidden, query, write_scores, episodic_keys, episodic_values,
           semantic_keys, semantic_values, episodic_ptr, semantic_ptr,
           Wk, bk, Wv, bv, Wr, br):
    eptr = episodic_ptr.astype(jnp.int32)
    sptr = semantic_ptr.astype(jnp.int32)

    ekT = jnp.swapaxes(episodic_keys, 1, 2)
    skT = jnp.swapaxes(semantic_keys, 1, 2)
    WkT = Wk.T
    WvT = Wv.T

    all_scores3, wvals3 = _score(
        write_scores, eptr, sptr, hidden, query, ekT, skT, WkT, bk, WvT, bv)
    all_scores = all_scores3.reshape(B, ES + SS)

    eslot = (eptr % ES).reshape(B, 1)
    sslot = (sptr % SS).reshape(B, 1)
    maskf = (write_scores > 0.0).astype(jnp.float32).reshape(B, 1)
    ew, ei, esub, sw, si, ssub = _topk_all(all_scores, eslot, sslot, maskf)

    eidx = ei[:, :K].reshape(B * K)
    sidx = si[:, :K].reshape(B * K)
    etab = jnp.swapaxes(episodic_values, 1, 2).reshape(B * VD * ES)
    stab = jnp.swapaxes(semantic_values, 1, 2).reshape(B * VD * SS)
    erows_w, srows_w = _sc_gather(etab, stab, eidx, sidx)

    er_t = erows_w.reshape(B, VD, K).transpose(2, 0, 1)
    sr_t = srows_w.reshape(B, VD, K).transpose(2, 0, 1)
    ew_t = ew[:, :K].T
    sw_t = sw[:, :K].T
    esub_t = esub[:, :K].T
    ssub_t = ssub[:, :K].T
    wvals = wvals3[:, 0, :]

    context = _combine(er_t, sr_t, ew_t, sw_t, esub_t, ssub_t,
                       wvals, Wr, br)
    return context, all_scores


# speedup vs baseline: 5.3131x; 1.7497x over previous
import functools

import jax
import jax.numpy as jnp
from jax import lax
from jax.experimental import pallas as pl
from jax.experimental.pallas import tpu as pltpu
from jax.experimental.pallas import tpu_sc as plsc

B, T, D = 32, 128, 1024
KD, VD = 64, 64
ES, SS = 16384, 4096
K = 32
LANES = 128
NEG = -3.0e38


def _score_body(ws_ref, eptr_ref, sptr_ref, hid_ref, q_ref, ek_ref,
                sk_ref, wk_ref, bk_ref, wv_ref, bv_ref,
                scores_ref, wvals_ref):
    b = pl.program_id(0)
    mask_b = ws_ref[b] > 0.0
    e_slot = lax.rem(eptr_ref[b], ES)
    s_slot = lax.rem(sptr_ref[b], SS)

    dn_t = (((1,), (1,)), ((), ()))
    dn_n = (((1,), (0,)), ((), ()))

    crep = jnp.mean(hid_ref[0], axis=0, keepdims=True)
    wkeys = lax.dot_general(crep, wk_ref[...], dn_t) + bk_ref[...][None, :]
    wvals = lax.dot_general(crep, wv_ref[...], dn_t) + bv_ref[...][None, :]
    q = lax.dot_general(q_ref[0], wk_ref[...], dn_t) + bk_ref[...][None, :]
    q_dot_wk = jnp.sum(q * wkeys)

    e_row = lax.dot_general(q, ek_ref[0], dn_n)
    col = lax.broadcasted_iota(jnp.int32, (1, ES), 1)
    e_row = jnp.where((col == e_slot) & mask_b, q_dot_wk, e_row)
    scores_ref[0, :, :ES] = e_row

    s_row = lax.dot_general(q, sk_ref[0], dn_n)
    scol = lax.broadcasted_iota(jnp.int32, (1, SS), 1)
    s_row = jnp.where((scol == s_slot) & mask_b, q_dot_wk, s_row)
    scores_ref[0, :, ES:] = s_row

    wvals_ref[0] = wvals


def _score(ws, eptr, sptr, hidden, query, ekT, skT, WkT, bk, WvT, bv):
    f32 = jnp.float32
    return pl.pallas_call(
        _score_body,
        grid=(B,),
        in_specs=[
            pl.BlockSpec(memory_space=pltpu.SMEM),
            pl.BlockSpec(memory_space=pltpu.SMEM),
            pl.BlockSpec(memory_space=pltpu.SMEM),
            pl.BlockSpec((1, T, D), lambda b: (b, 0, 0)),
            pl.BlockSpec((1, 1, D), lambda b: (b, 0, 0)),
            pl.BlockSpec((1, KD, ES), lambda b: (b, 0, 0)),
            pl.BlockSpec((1, KD, SS), lambda b: (b, 0, 0)),
            pl.BlockSpec((KD, D), lambda b: (0, 0)),
            pl.BlockSpec((KD,), lambda b: (0,)),
            pl.BlockSpec((KD, D), lambda b: (0, 0)),
            pl.BlockSpec((KD,), lambda b: (0,)),
        ],
        out_specs=[
            pl.BlockSpec((1, 1, ES + SS), lambda b: (b, 0, 0)),
            pl.BlockSpec((1, 1, VD), lambda b: (b, 0, 0)),
        ],
        out_shape=[
            jax.ShapeDtypeStruct((B, 1, ES + SS), f32),
            jax.ShapeDtypeStruct((B, 1, VD), f32),
        ],
        compiler_params=pltpu.CompilerParams(
            dimension_semantics=("arbitrary",)),
    )(ws, eptr, sptr, hidden, query.reshape(B, 1, D), ekT, skT,
      WkT, bk, WvT, bv)


def _topk_all_body(sc_ref, eslot_ref, sslot_ref, maskf_ref,
                   ew_ref, ei_ref, esub_ref, sw_ref, si_ref, ssub_ref):
    lane = lax.broadcasted_iota(jnp.int32, (B, LANES), 1)
    row = lax.broadcasted_iota(jnp.int32, (B, LANES), 0)
    maskb = maskf_ref[...] > 0.5
    big = jnp.int32(0x7FFFFFFF)

    def extract(s, n, slot, base_mul, w_ref, i_ref, sub_ref):
        colio = lax.broadcasted_iota(jnp.int32, (B, n), 1)
        vals = jnp.full((B, LANES), NEG, dtype=jnp.float32)
        idxm = jnp.zeros((B, LANES), dtype=jnp.int32)
        for k in range(K):
            m = jnp.max(s, axis=1, keepdims=True)
            cand = jnp.where(s == m, colio, big)
            am = jnp.min(cand, axis=1, keepdims=True)
            s = jnp.where(cand == am, NEG, s)
            vals = jnp.where(lane == k, m, vals)
            idxm = jnp.where(lane == k, am, idxm)
        valid = (lane < K).astype(jnp.float32)
        w = jnp.exp(vals - vals[:, 0:1]) * valid
        w_ref[...] = w / jnp.sum(w, axis=1, keepdims=True)
        i_ref[...] = idxm + row * base_mul
        sub_ref[...] = jnp.where(
            (idxm == slot) & maskb & (lane < K), 1.0, 0.0)

    extract(sc_ref[:, :ES], ES, eslot_ref[...], ES, ew_ref, ei_ref, esub_ref)
    extract(sc_ref[:, ES:], SS, sslot_ref[...], SS, sw_ref, si_ref, ssub_ref)


def _topk_all(all_scores, eslot, sslot, maskf):
    f32 = jnp.float32
    i32 = jnp.int32
    sf = jax.ShapeDtypeStruct((B, LANES), f32)
    si_ = jax.ShapeDtypeStruct((B, LANES), i32)
    return pl.pallas_call(
        _topk_all_body,
        out_shape=[sf, si_, sf, sf, si_, sf],
    )(all_scores, eslot, sslot, maskf)


NW = 32
RPW = (B * K) // NW
EPW = RPW * VD

EB = ES.bit_length() - 1
SB = SS.bit_length() - 1


def _build_addrs(idx_v, addr_v, slot_bits):
    for g in range(RPW // 16):
        gv = idx_v[pl.ds(g * 16, 16)]
        b8 = (gv >> slot_bits) << 3
        slot = gv & ((1 << slot_bits) - 1)
        sb = slot >> 7
        sl = slot & 127
        for fb in range(VD // 8):
            mid = (((((b8 + fb) << (slot_bits - 7)) + sb) << 10) + sl)
            for f8 in range(8):
                p = (fb * 8 + f8) * RPW + g * 16
                addr_v[p // LANES, pl.ds(p % LANES, 16)] = mid + (f8 << 7)


def _gather_rows(tab_ref, addr_v, rows_v, sem):
    copies = []
    for j in range(EPW // LANES):
        copies.append(pltpu.async_copy(
            tab_ref.at[addr_v.at[j]], rows_v.at[pl.ds(j * LANES, LANES)], sem))
    for c in copies:
        c.wait()


def _sc_gather_body(etab_ref, stab_ref, eidx_ref, sidx_ref,
                    erows_ref, srows_ref, idx_v, addr_v, rows_v, sem):
    wid = lax.axis_index("s") * 2 + lax.axis_index("c")
    base = wid * RPW

    pltpu.sync_copy(eidx_ref.at[pl.ds(base, RPW)], idx_v)
    _build_addrs(idx_v, addr_v, EB)
    _gather_rows(etab_ref, addr_v, rows_v, sem)
    pltpu.sync_copy(rows_v, erows_ref.at[wid])

    pltpu.sync_copy(sidx_ref.at[pl.ds(base, RPW)], idx_v)
    _build_addrs(idx_v, addr_v, SB)
    _gather_rows(stab_ref, addr_v, rows_v, sem)
    pltpu.sync_copy(rows_v, srows_ref.at[wid])


def _sc_gather(etab, stab, eidx, sidx):
    mesh = plsc.VectorSubcoreMesh(core_axis_name="c", subcore_axis_name="s")
    fn = functools.partial(
        pl.kernel,
        out_type=[
            jax.ShapeDtypeStruct((NW, EPW), jnp.float32),
            jax.ShapeDtypeStruct((NW, EPW), jnp.float32),
        ],
        mesh=mesh,
        scratch_types=[
            pltpu.VMEM((RPW,), jnp.int32),
            pltpu.VMEM((EPW // LANES, LANES), jnp.int32),
            pltpu.VMEM((EPW,), jnp.float32),
            pltpu.SemaphoreType.DMA,
        ],
    )(_sc_gather_body)
    return fn(etab, stab, eidx, sidx)


def _combine_body(er_ref, sr_ref, ew_ref, sw_ref, esub_ref, ssub_ref,
                  wv_ref, Wr_ref, br_ref, out_ref):
    wv = wv_ref[...]
    acc = jnp.zeros((B, VD), dtype=jnp.float32)
    for k in range(K):
        er = er_ref[k]
        sr = sr_ref[k]
        es = esub_ref[k][:, None]
        ss_ = ssub_ref[k][:, None]
        ewc = ew_ref[k][:, None]
        swc = sw_ref[k][:, None]
        acc = acc + (er * (1.0 - es) + wv * es) * ewc
        acc = acc + (sr * (1.0 - ss_) + wv * ss_) * swc
    out_ref[...] = jnp.dot(acc, Wr_ref[...]) + br_ref[...][None, :]


def _combine(er_t, sr_t, ew_t, sw_t, esub_t, ssub_t, wvals, Wr, br):
    return pl.pallas_call(
        _combine_body,
        out_shape=jax.ShapeDtypeStruct((B, D), jnp.float32),
    )(er_t, sr_t, ew_t, sw_t, esub_t, ssub_t, wvals, Wr, br)


def kernel(hidden, query, write_scores, episodic_keys, episodic_values,
           semantic_keys, semantic_values, episodic_ptr, semantic_ptr,
           Wk, bk, Wv, bv, Wr, br):
    eptr = episodic_ptr.astype(jnp.int32)
    sptr = semantic_ptr.astype(jnp.int32)

    ekT = jnp.swapaxes(episodic_keys, 1, 2)
    skT = jnp.swapaxes(semantic_keys, 1, 2)
    WkT = Wk.T
    WvT = Wv.T

    all_scores3, wvals3 = _score(
        write_scores, eptr, sptr, hidden, query, ekT, skT, WkT, bk, WvT, bv)
    all_scores = all_scores3.reshape(B, ES + SS)

    eslot = (eptr % ES).reshape(B, 1)
    sslot = (sptr % SS).reshape(B, 1)
    maskf = (write_scores > 0.0).astype(jnp.float32).reshape(B, 1)
    ew, ei, esub, sw, si, ssub = _topk_all(all_scores, eslot, sslot, maskf)

    eidx = ei[:, :K].reshape(B * K)
    sidx = si[:, :K].reshape(B * K)
    etab = jnp.swapaxes(episodic_values, 1, 2) \
        .reshape(B, VD // 8, 8, ES // 128, 128) \
        .transpose(0, 1, 3, 2, 4).reshape(B * VD * ES)
    stab = jnp.swapaxes(semantic_values, 1, 2) \
        .reshape(B, VD // 8, 8, SS // 128, 128) \
        .transpose(0, 1, 3, 2, 4).reshape(B * VD * SS)
    erows_w, srows_w = _sc_gather(etab, stab, eidx, sidx)

    er_t = erows_w.reshape(B, VD, K).transpose(2, 0, 1)
    sr_t = srows_w.reshape(B, VD, K).transpose(2, 0, 1)
    ew_t = ew[:, :K].T
    sw_t = sw[:, :K].T
    esub_t = esub[:, :K].T
    ssub_t = ssub[:, :K].T
    wvals = wvals3[:, 0, :]

    context = _combine(er_t, sr_t, ew_t, sw_t, esub_t, ssub_t,
                       wvals, Wr, br)
    return context, all_scores
